# Initial kernel scaffold; baseline (speedup 1.0000x reference)
import jax
import jax.numpy as jnp
from jax.experimental import pallas as pl


def kernel(x0, x1, x2, x3, edge_index0, edge_index1, edge_index2, edge_index3, n_owned, W, b):
    raise NotImplementedError("write your pallas kernel here")


# speedup vs baseline: 2.6768x; 2.6768x over previous
import functools

import jax
import jax.numpy as jnp
from jax import lax
from jax.experimental import pallas as pl
from jax.experimental.pallas import tpu as pltpu
from jax.experimental.pallas import tpu_sc as plsc

N_NODES = 10000
N_EDGES = 320000
D = 128

NC = 2
NS = 16
NW = NC * NS

CHUNK = 128
CPW = 80
EPW = CHUNK * CPW
E_PAD = EPW * NW
ACC_ROWS = 10240
ROWS_PER_TILE = ACC_ROWS // NS
ZB = 32
SLAB = 40
N_OWN = 8000


def _sc_body(x0, x1, x2, x3, s0, s1, s2, s3, d0, d1, d2, d3, out,
             acc, src_v, dst_v, rows0, rows1, zbuf, sem0, sem1):
    c = lax.axis_index("c")
    s = lax.axis_index("s")
    wid = c * NS + s

    zv = jnp.zeros((16,), jnp.float32)

    @pl.loop(0, ZB)
    def _zero_zbuf(i):
        for j in range(D // 16):
            zbuf[i, pl.ds(j * 16, 16)] = zv

    xs = (x0, x1, x2, x3)
    ss = (s0, s1, s2, s3)
    ds = (d0, d1, d2, d3)

    for g in range(4):
        xg, sg, dg = xs[g], ss[g], ds[g]

        r0 = s * ROWS_PER_TILE
        for k in range(ROWS_PER_TILE // ZB):
            pltpu.sync_copy(zbuf, acc.at[pl.ds(r0 + k * ZB, ZB)])

        plsc.subcore_barrier()

        for sl in range(CPW // SLAB):
            pltpu.sync_copy(sg.at[pl.ds(wid * CPW + sl * SLAB, SLAB)], src_v)
            pltpu.sync_copy(dg.at[pl.ds(wid * CPW + sl * SLAB, SLAB)], dst_v)

            pltpu.async_copy(xg.at[src_v.at[0]], rows0, sem0)
            pltpu.async_copy(xg.at[src_v.at[1]], rows1, sem1)

            @pl.loop(0, SLAB // 2 - 1)
            def _edge_loop(t2):
                c0 = t2 * 2
                pltpu.make_async_copy(xg.at[src_v.at[0]], rows0, sem0).wait()
                pltpu.sync_copy(rows0, acc.at[dst_v.at[c0]], add=True)
                pltpu.async_copy(xg.at[src_v.at[c0 + 2]], rows0, sem0)
                pltpu.make_async_copy(xg.at[src_v.at[0]], rows1, sem1).wait()
                pltpu.sync_copy(rows1, acc.at[dst_v.at[c0 + 1]], add=True)
                pltpu.async_copy(xg.at[src_v.at[c0 + 3]], rows1, sem1)

            pltpu.make_async_copy(xg.at[src_v.at[0]], rows0, sem0).wait()
            pltpu.sync_copy(rows0, acc.at[dst_v.at[SLAB - 2]], add=True)
            pltpu.make_async_copy(xg.at[src_v.at[0]], rows1, sem1).wait()
            pltpu.sync_copy(rows1, acc.at[dst_v.at[SLAB - 1]], add=True)

        plsc.subcore_barrier()

        pltpu.sync_copy(acc.at[pl.ds(r0, ROWS_PER_TILE)],
                        out.at[g, c, pl.ds(r0, ROWS_PER_TILE)])

        plsc.subcore_barrier()


_sc_segment_sum = functools.partial(
    pl.kernel,
    out_type=jax.ShapeDtypeStruct((4, NC, ACC_ROWS, D), jnp.float32),
    mesh=plsc.VectorSubcoreMesh(core_axis_name="c", subcore_axis_name="s",
                                num_cores=NC, num_subcores=NS),
    scratch_types=[
        pltpu.VMEM_SHARED((ACC_ROWS, D), jnp.float32),
        pltpu.VMEM((SLAB, CHUNK), jnp.int32),
        pltpu.VMEM((SLAB, CHUNK), jnp.int32),
        pltpu.VMEM((CHUNK, D), jnp.float32),
        pltpu.VMEM((CHUNK, D), jnp.float32),
        pltpu.VMEM((ZB, D), jnp.float32),
        pltpu.SemaphoreType.DMA,
        pltpu.SemaphoreType.DMA,
    ],
)(_sc_body)


BM = 1000


def _mm_body(x_ref, p0_ref, p1_ref, w1_ref, w2_ref, b_ref, o_ref):
    ng = p0_ref[...] + p1_ref[...]
    o_ref[...] = (
        jnp.dot(x_ref[...], w1_ref[...], precision=jax.lax.Precision.HIGHEST,
                preferred_element_type=jnp.float32)
        + jnp.dot(ng, w2_ref[...], precision=jax.lax.Precision.HIGHEST,
                  preferred_element_type=jnp.float32)
        + b_ref[...]
    )


def _tc_linear(x, p0, p1, w1t, w2t, b2):
    return pl.pallas_call(
        _mm_body,
        grid=(N_OWN // BM,),
        in_specs=[
            pl.BlockSpec((BM, D), lambda i: (i, 0)),
            pl.BlockSpec((BM, D), lambda i: (i, 0)),
            pl.BlockSpec((BM, D), lambda i: (i, 0)),
            pl.BlockSpec((D, D), lambda i: (0, 0)),
            pl.BlockSpec((D, D), lambda i: (0, 0)),
            pl.BlockSpec((1, D), lambda i: (0, 0)),
        ],
        out_specs=pl.BlockSpec((BM, D), lambda i: (i, 0)),
        out_shape=jax.ShapeDtypeStruct((N_OWN, D), jnp.float32),
    )(x, p0, p1, w1t, w2t, b2)


def kernel(x0, x1, x2, x3, edge_index0, edge_index1, edge_index2, edge_index3,
           n_owned, W, b):
    pad = E_PAD - N_EDGES
    srcs, dsts = [], []
    for ei in (edge_index0, edge_index1, edge_index2, edge_index3):
        src = jnp.concatenate([ei[0], jnp.zeros((pad,), jnp.int32)])
        dst = jnp.concatenate([ei[1], jnp.full((pad,), N_NODES, jnp.int32)])
        srcs.append(src.reshape(NW * CPW, CHUNK))
        dsts.append(dst.reshape(NW * CPW, CHUNK))

    partials = _sc_segment_sum(x0, x1, x2, x3, *srcs, *dsts)

    start = n_owned - N_OWN
    w1t = W[:, :D].T
    w2t = W[:, D:].T
    b2 = b[None, :]
    outs = []
    for g, xg in enumerate((x0, x1, x2, x3)):
        x_own = lax.dynamic_slice_in_dim(xg, start, N_OWN, axis=0)
        p0 = lax.dynamic_slice_in_dim(partials[g, 0], start, N_OWN, axis=0)
        p1 = lax.dynamic_slice_in_dim(partials[g, 1], start, N_OWN, axis=0)
        outs.append(_tc_linear(x_own, p0, p1, w1t, w2t, b2))
    return tuple(outs)
